# baseline (speedup 1.0000x reference)
import jax
import jax.numpy as jnp
from jax import lax
from jax.experimental import pallas as pl
from jax.experimental.pallas import tpu as pltpu
from jax.experimental.pallas import tpu_sc as plsc

N = 4096
C = 128
K = 16
RT = 512
GRID = N // RT
NW = 32
NPT = N // NW
GCH = 8
NCH = NPT // GCH
F32 = jnp.float32


def _embed_body(x_ref, w_ref, b_ref, o_ref):
    o_ref[...] = jnp.maximum(
        jnp.dot(x_ref[...], w_ref[...], preferred_element_type=F32)
        + b_ref[...], 0.0)


def _embed(x, w, b):
    cin = x.shape[1]
    return pl.pallas_call(
        _embed_body,
        grid=(GRID,),
        in_specs=[
            pl.BlockSpec((RT, cin), lambda i: (i, 0)),
            pl.BlockSpec((cin, C), lambda i: (0, 0)),
            pl.BlockSpec((1, C), lambda i: (0, 0)),
        ],
        out_specs=pl.BlockSpec((RT, C), lambda i: (i, 0)),
        out_shape=jax.ShapeDtypeStruct((N, C), F32),
    )(x, w, b.reshape(1, C))


def _knn_body(x_ref, xt_ref, xsqc_ref, xsqr_ref, idx_ref):
    g = jnp.dot(x_ref[...], xt_ref[...], preferred_element_type=F32)
    d = (xsqc_ref[...] - 2.0 * g) + xsqr_ref[...]
    q = N // 4
    iota = lax.broadcasted_iota(jnp.int32, (RT, q), 1)
    vals = [d[:, i * q:(i + 1) * q] for i in range(4)]
    idxs = [iota + (i * q) for i in range(4)]
    for i, j in [(0, 1), (2, 3), (0, 2), (1, 3), (1, 2)]:
        xi, xj, ii, ij = vals[i], vals[j], idxs[i], idxs[j]
        swap = (xj < xi) | ((xj == xi) & (ij < ii))
        vals[i] = jnp.where(swap, xj, xi)
        vals[j] = jnp.where(swap, xi, xj)
        idxs[i] = jnp.where(swap, ij, ii)
        idxs[j] = jnp.where(swap, ii, ij)
    r0, r1, r2, r3 = vals
    i0, i1, i2, i3 = idxs
    inf = jnp.float32(jnp.inf)
    cols = []
    for _ in range(K):
        m = jnp.min(r0, axis=1, keepdims=True)
        am = jnp.min(jnp.where(r0 == m, i0, N), axis=1, keepdims=True)
        cols.append(am)
        pos = i0 == am
        r0 = jnp.where(pos, r1, r0)
        i0 = jnp.where(pos, i1, i0)
        r1 = jnp.where(pos, r2, r1)
        i1 = jnp.where(pos, i2, i1)
        r2 = jnp.where(pos, r3, r2)
        i2 = jnp.where(pos, i3, i2)
        r3 = jnp.where(pos, inf, r3)
    idx_ref[...] = jnp.concatenate(cols, axis=1)


def _knn(x, xsq):
    return pl.pallas_call(
        _knn_body,
        grid=(GRID,),
        in_specs=[
            pl.BlockSpec((RT, C), lambda i: (i, 0)),
            pl.BlockSpec((C, N), lambda i: (0, 0)),
            pl.BlockSpec((RT, 1), lambda i: (i, 0)),
            pl.BlockSpec((1, N), lambda i: (0, 0)),
        ],
        out_specs=pl.BlockSpec((RT, K), lambda i: (i, 0)),
        out_shape=jax.ShapeDtypeStruct((N, K), jnp.int32),
    )(x, x.T, xsq.reshape(N, 1), xsq.reshape(1, N))


def _edge_body(x_ref, xj_ref, wt_ref, wb_ref, b_ref, r_ref, o_ref):
    xi = x_ref[...]
    a = jnp.dot(xi, wt_ref[...], preferred_element_type=F32) + b_ref[...]
    acc = None
    for j in range(K):
        e = xj_ref[:, j, :] - xi
        h = jnp.maximum(
            a + jnp.dot(e, wb_ref[...], preferred_element_type=F32), 0.0)
        acc = h if acc is None else jnp.maximum(acc, h)
    o_ref[...] = acc + r_ref[...]


def _edge(x, xj, wt, wb, b, res):
    return pl.pallas_call(
        _edge_body,
        grid=(GRID,),
        in_specs=[
            pl.BlockSpec((RT, C), lambda i: (i, 0)),
            pl.BlockSpec((RT, K, C), lambda i: (i, 0, 0)),
            pl.BlockSpec((C, C), lambda i: (0, 0)),
            pl.BlockSpec((C, C), lambda i: (0, 0)),
            pl.BlockSpec((1, C), lambda i: (0, 0)),
            pl.BlockSpec((RT, C), lambda i: (i, 0)),
        ],
        out_specs=pl.BlockSpec((RT, C), lambda i: (i, 0)),
        out_shape=jax.ShapeDtypeStruct((N, C), F32),
    )(x, xj, wt, wb, b.reshape(1, C), res)


def _sc_gather_body(idx_hbm, x_hbm, out_hbm, idxv, rows, sem):
    wid = lax.axis_index("s") * 2 + lax.axis_index("c")
    pltpu.sync_copy(idx_hbm.at[wid], idxv)

    def chunk(g, _):
        pltpu.async_copy(x_hbm.at[idxv.at[g]], rows, sem).wait()
        base = (wid * NPT + g * GCH) * K
        pltpu.sync_copy(rows, out_hbm.at[pl.ds(base, GCH * K)])
        return 0

    lax.fori_loop(0, NCH, chunk, 0)


def _sc_gather(idx, x):
    mesh = plsc.VectorSubcoreMesh(core_axis_name="c", subcore_axis_name="s")
    k = pl.kernel(
        _sc_gather_body,
        out_type=jax.ShapeDtypeStruct((N * K, C), F32),
        mesh=mesh,
        scratch_types=[
            pltpu.VMEM((NCH, GCH * K), jnp.int32),
            pltpu.VMEM((GCH * K, C), F32),
            pltpu.SemaphoreType.DMA,
        ],
    )
    return k(idx.reshape(NW, NCH, GCH * K), x)


def kernel(inputs, W_emb, b_emb, W0, b0, W1, b1, W2, b2, W3, b3):
    x = _embed(inputs, W_emb, b_emb)
    feas = []
    zero = jnp.zeros((N, C), F32)
    for i, (W, b) in enumerate([(W0, b0), (W1, b1), (W2, b2), (W3, b3)]):
        xsq = jnp.sum(x * x, axis=-1)
        idx = _knn(x, xsq)
        xj = _sc_gather(idx, x).reshape(N, K, C)
        res = zero if i == 0 else x
        x = _edge(x, xj, W[:C, :], W[C:, :], b, res)
        feas.append(x)
    return jnp.concatenate(feas, axis=-1)


# speedup vs baseline: 1.0005x; 1.0005x over previous
import functools

import jax
import jax.numpy as jnp
from jax import lax
from jax.experimental import pallas as pl
from jax.experimental.pallas import tpu as pltpu
from jax.experimental.pallas import tpu_sc as plsc

N = 4096
C = 128
K = 16
RT = 512
GRID = N // RT
NW = 32
NPT = N // NW
GCH = 8
NCH = NPT // GCH
F32 = jnp.float32


def _embed_body(x_ref, w_ref, b_ref, o_ref):
    o_ref[...] = jnp.maximum(
        jnp.dot(x_ref[...], w_ref[...], preferred_element_type=F32)
        + b_ref[...], 0.0)


def _embed(x, w, b):
    cin = x.shape[1]
    return pl.pallas_call(
        _embed_body,
        grid=(GRID,),
        in_specs=[
            pl.BlockSpec((RT, cin), lambda i: (i, 0)),
            pl.BlockSpec((cin, C), lambda i: (0, 0)),
            pl.BlockSpec((1, C), lambda i: (0, 0)),
        ],
        out_specs=pl.BlockSpec((RT, C), lambda i: (i, 0)),
        out_shape=jax.ShapeDtypeStruct((N, C), F32),
    )(x, w, b.reshape(1, C))


NH = N // 2


def _knn_body(x_ref, xt_ref, xsqc_ref, xsqr_ref, idx_ref):
    g = jnp.dot(x_ref[...], xt_ref[...], preferred_element_type=F32)
    d = (xsqc_ref[...] - 2.0 * g) + xsqr_ref[...]
    q = N // 4
    iota = lax.broadcasted_iota(jnp.int32, (RT, q), 1)
    vals = [d[:, i * q:(i + 1) * q] for i in range(4)]
    idxs = [iota + (i * q) for i in range(4)]
    for i, j in [(0, 1), (2, 3), (0, 2), (1, 3), (1, 2)]:
        xi, xj, ii, ij = vals[i], vals[j], idxs[i], idxs[j]
        swap = (xj < xi) | ((xj == xi) & (ij < ii))
        vals[i] = jnp.where(swap, xj, xi)
        vals[j] = jnp.where(swap, xi, xj)
        idxs[i] = jnp.where(swap, ij, ii)
        idxs[j] = jnp.where(swap, ii, ij)
    r0, r1, r2, r3 = vals
    i0, i1, i2, i3 = idxs
    inf = jnp.float32(jnp.inf)
    cols = []
    for _ in range(K):
        m = jnp.min(r0, axis=1, keepdims=True)
        am = jnp.min(jnp.where(r0 == m, i0, N), axis=1, keepdims=True)
        cols.append(am)
        pos = i0 == am
        r0 = jnp.where(pos, r1, r0)
        i0 = jnp.where(pos, i1, i0)
        r1 = jnp.where(pos, r2, r1)
        i1 = jnp.where(pos, i2, i1)
        r2 = jnp.where(pos, r3, r2)
        i2 = jnp.where(pos, i3, i2)
        r3 = jnp.where(pos, inf, r3)
    idx_ref[...] = jnp.concatenate(cols, axis=1)


def _knn(xh, xft, xsqh, xsqf):
    nl = xh.shape[0]
    return pl.pallas_call(
        _knn_body,
        grid=(nl // RT,),
        in_specs=[
            pl.BlockSpec((RT, C), lambda i: (i, 0)),
            pl.BlockSpec((C, N), lambda i: (0, 0)),
            pl.BlockSpec((RT, 1), lambda i: (i, 0)),
            pl.BlockSpec((1, N), lambda i: (0, 0)),
        ],
        out_specs=pl.BlockSpec((RT, K), lambda i: (i, 0)),
        out_shape=jax.ShapeDtypeStruct((nl, K), jnp.int32),
    )(xh, xft, xsqh.reshape(nl, 1), xsqf.reshape(1, N))


def _edge_body(x_ref, xj_ref, wt_ref, wb_ref, b_ref, r_ref, o_ref):
    xi = x_ref[...]
    a = jnp.dot(xi, wt_ref[...], preferred_element_type=F32) + b_ref[...]
    acc = None
    for j in range(K):
        e = xj_ref[:, j, :] - xi
        h = jnp.maximum(
            a + jnp.dot(e, wb_ref[...], preferred_element_type=F32), 0.0)
        acc = h if acc is None else jnp.maximum(acc, h)
    o_ref[...] = acc + r_ref[...]


def _edge(x, xj, wt, wb, b, res):
    return pl.pallas_call(
        _edge_body,
        grid=(x.shape[0] // RT,),
        in_specs=[
            pl.BlockSpec((RT, C), lambda i: (i, 0)),
            pl.BlockSpec((RT, K, C), lambda i: (i, 0, 0)),
            pl.BlockSpec((C, C), lambda i: (0, 0)),
            pl.BlockSpec((C, C), lambda i: (0, 0)),
            pl.BlockSpec((1, C), lambda i: (0, 0)),
            pl.BlockSpec((RT, C), lambda i: (i, 0)),
        ],
        out_specs=pl.BlockSpec((RT, C), lambda i: (i, 0)),
        out_shape=jax.ShapeDtypeStruct((x.shape[0], C), F32),
    )(x, xj, wt, wb, b.reshape(1, C), res)


def _sc_gather_body(npt, nch, idx_hbm, x_hbm, out_hbm, idxv, rows, sem):
    wid = lax.axis_index("s") * 2 + lax.axis_index("c")
    pltpu.sync_copy(idx_hbm.at[wid], idxv)

    def chunk(g, _):
        pltpu.async_copy(x_hbm.at[idxv.at[g]], rows, sem).wait()
        base = (wid * npt + g * GCH) * K
        pltpu.sync_copy(rows, out_hbm.at[pl.ds(base, GCH * K)])
        return 0

    lax.fori_loop(0, nch, chunk, 0)


def _sc_gather(idx, x):
    nl = idx.shape[0]
    npt = nl // NW
    nch = npt // GCH
    mesh = plsc.VectorSubcoreMesh(core_axis_name="c", subcore_axis_name="s")
    k = pl.kernel(
        functools.partial(_sc_gather_body, npt, nch),
        out_type=jax.ShapeDtypeStruct((nl * K, C), F32),
        mesh=mesh,
        scratch_types=[
            pltpu.VMEM((nch, GCH * K), jnp.int32),
            pltpu.VMEM((GCH * K, C), F32),
            pltpu.SemaphoreType.DMA,
        ],
    )
    return k(idx.reshape(NW, nch, GCH * K), x)


def kernel(inputs, W_emb, b_emb, W0, b0, W1, b1, W2, b2, W3, b3):
    x = _embed(inputs, W_emb, b_emb)
    feas = []
    zero = jnp.zeros((NH, C), F32)
    for i, (W, b) in enumerate([(W0, b0), (W1, b1), (W2, b2), (W3, b3)]):
        xft = x.T
        xsq = jnp.sum(x * x, axis=-1)
        xa, xb = x[:NH], x[NH:]
        idxa = _knn(xa, xft, xsq[:NH], xsq)
        xja = _sc_gather(idxa, x)
        idxb = _knn(xb, xft, xsq[NH:], xsq)
        xjb = _sc_gather(idxb, x)
        wt, wb_ = W[:C, :], W[C:, :]
        ya = _edge(xa, xja.reshape(NH, K, C), wt, wb_, b,
                   zero if i == 0 else xa)
        yb = _edge(xb, xjb.reshape(NH, K, C), wt, wb_, b,
                   zero if i == 0 else xb)
        x = jnp.concatenate([ya, yb], axis=0)
        feas.append(x)
    return jnp.concatenate(feas, axis=-1)
